# baseline (speedup 1.0000x reference)
import functools

import jax
import jax.numpy as jnp
from jax import lax
from jax.experimental import pallas as pl
from jax.experimental.pallas import tpu as pltpu
from jax.experimental.pallas import tpu_sc as plsc

_K = 1024
_D = 256
_B = 8
_T = 576
_N = _B * _T

_NW = 32
_BPW = _N // _NW
_GCH = 72


def _tc_body(x_ref, e_ref, idx_ref, hist_ref, loss_ref):
    b = pl.program_id(0)
    x = x_ref[0]
    e = e_ref[...]
    scores = lax.dot_general(e, x, (((1,), (0,)), ((), ())),
                             preferred_element_type=jnp.float32)
    e2 = jnp.sum(e * e, axis=1, keepdims=True)
    x2 = jnp.sum(x * x, axis=0, keepdims=True)
    dist = x2 - 2.0 * scores + e2
    idx = jnp.argmin(dist, axis=0)
    minv = jnp.min(dist, axis=0)
    idx_ref[0, 0, :] = idx

    iota_k = lax.broadcasted_iota(jnp.int32, (_K, _T), 0)
    eq = (iota_k == idx[None, :]).astype(jnp.float32)
    hist_part = jnp.sum(eq, axis=1, keepdims=True)
    loss_part = jnp.sum(minv)

    @pl.when(b == 0)
    def _init():
        hist_ref[...] = jnp.zeros_like(hist_ref)
        loss_ref[0, 0] = 0.0

    hist_ref[...] += hist_part
    loss_ref[0, 0] += loss_part


def _tc_call(inputs, embed_weight):
    return pl.pallas_call(
        _tc_body,
        grid=(_B,),
        in_specs=[
            pl.BlockSpec((1, _D, _T), lambda b: (b, 0, 0)),
            pl.BlockSpec((_K, _D), lambda b: (0, 0)),
        ],
        out_specs=(
            pl.BlockSpec((1, 1, _T), lambda b: (b, 0, 0)),
            pl.BlockSpec((_K, 1), lambda b: (0, 0)),
            pl.BlockSpec((1, 1), lambda b: (0, 0), memory_space=pltpu.SMEM),
        ),
        out_shape=(
            jax.ShapeDtypeStruct((_B, 1, _T), jnp.int32),
            jax.ShapeDtypeStruct((_K, 1), jnp.float32),
            jax.ShapeDtypeStruct((1, 1), jnp.float32),
        ),
    )(inputs, embed_weight)


@functools.cache
def _get_sc_gather():
    mesh = plsc.VectorSubcoreMesh(core_axis_name="c", subcore_axis_name="s")

    @functools.partial(
        pl.kernel,
        mesh=mesh,
        out_type=jax.ShapeDtypeStruct((_N, _D), jnp.float32),
        scratch_types=[
            pltpu.VMEM((_BPW,), jnp.int32),
            pltpu.VMEM((_BPW, _D), jnp.float32),
            pltpu.SemaphoreType.DMA,
        ],
    )
    def _sc_gather(table_hbm, idx_hbm, out_hbm, idx_v, rows_v, sem):
        wid = lax.axis_index("s") * 2 + lax.axis_index("c")
        base = wid * _BPW
        pltpu.sync_copy(idx_hbm.at[pl.ds(base, _BPW)], idx_v)
        c0 = pltpu.async_copy(table_hbm.at[idx_v.at[pl.ds(0, _GCH)]],
                              rows_v.at[pl.ds(0, _GCH)], sem)
        c1 = pltpu.async_copy(table_hbm.at[idx_v.at[pl.ds(_GCH, _GCH)]],
                              rows_v.at[pl.ds(_GCH, _GCH)], sem)
        c0.wait()
        c1.wait()
        pltpu.sync_copy(rows_v, out_hbm.at[pl.ds(base, _BPW)])

    return _sc_gather


def kernel(inputs, embed_weight, training):
    idx3, hist, loss_sum = _tc_call(inputs, embed_weight)
    idx_flat = idx3.reshape(_N)
    qflat = _get_sc_gather()(embed_weight, idx_flat)
    quantized_out = jnp.transpose(qflat.reshape(_B, _T, _D), (0, 2, 1))
    mse = loss_sum[0, 0] / (_N * _D)
    loss = mse * 1.2
    cmt_loss = mse
    avg_probs = hist[:, 0] / _N
    perplexity = jnp.exp(-jnp.sum(avg_probs * jnp.log(avg_probs + 1e-10)))
    enc_idx = idx_flat.reshape(_B, _T, 1).astype(jnp.int64)
    return (quantized_out, loss, cmt_loss, enc_idx, perplexity)


# speedup vs baseline: 1.0301x; 1.0301x over previous
import functools

import jax
import jax.numpy as jnp
from jax import lax
from jax.experimental import pallas as pl
from jax.experimental.pallas import tpu as pltpu
from jax.experimental.pallas import tpu_sc as plsc

_K = 1024
_D = 256
_B = 8
_T = 576
_N = _B * _T

_NW = 32
_BPW = _N // _NW
_GCH = 72


def _tc_body(x_ref, e_ref, idx_ref, loss_ref, cmt_ref, perp_ref,
             hist_acc, loss_acc):
    b = pl.program_id(0)
    x = x_ref[0]
    e = e_ref[...]
    scores = lax.dot_general(e, x, (((1,), (0,)), ((), ())),
                             preferred_element_type=jnp.float32)
    e2 = jnp.sum(e * e, axis=1, keepdims=True)
    x2 = jnp.sum(x * x, axis=0, keepdims=True)
    dist = x2 - 2.0 * scores + e2
    idx = jnp.argmin(dist, axis=0)
    minv = jnp.min(dist, axis=0)
    idx_ref[0, 0, :] = idx

    iota_k = lax.broadcasted_iota(jnp.int32, (_K, _T), 0)
    eq = (iota_k == idx[None, :]).astype(jnp.float32)
    hist_part = jnp.sum(eq, axis=1, keepdims=True)
    loss_part = jnp.sum(minv)

    @pl.when(b == 0)
    def _init():
        hist_acc[...] = jnp.zeros_like(hist_acc)
        loss_acc[0] = 0.0

    hist_acc[...] += hist_part
    loss_acc[0] += loss_part

    @pl.when(b == _B - 1)
    def _finish():
        mse = loss_acc[0] / (_N * _D)
        cmt_ref[0, 0] = mse
        loss_ref[0, 0] = mse * 1.2
        avg = hist_acc[...] / _N
        ent = jnp.sum(avg * jnp.log(avg + 1e-10))
        perp_ref[0, 0] = jnp.exp(-ent)


def _tc_call(inputs, embed_weight):
    return pl.pallas_call(
        _tc_body,
        grid=(_B,),
        in_specs=[
            pl.BlockSpec((1, _D, _T), lambda b: (b, 0, 0)),
            pl.BlockSpec((_K, _D), lambda b: (0, 0)),
        ],
        out_specs=(
            pl.BlockSpec((1, 1, _T), lambda b: (b, 0, 0)),
            pl.BlockSpec((1, 1), lambda b: (0, 0), memory_space=pltpu.SMEM),
            pl.BlockSpec((1, 1), lambda b: (0, 0), memory_space=pltpu.SMEM),
            pl.BlockSpec((1, 1), lambda b: (0, 0), memory_space=pltpu.SMEM),
        ),
        out_shape=(
            jax.ShapeDtypeStruct((_B, 1, _T), jnp.int32),
            jax.ShapeDtypeStruct((1, 1), jnp.float32),
            jax.ShapeDtypeStruct((1, 1), jnp.float32),
            jax.ShapeDtypeStruct((1, 1), jnp.float32),
        ),
        scratch_shapes=[
            pltpu.VMEM((_K, 1), jnp.float32),
            pltpu.SMEM((1,), jnp.float32),
        ],
    )(inputs, embed_weight)


@functools.cache
def _get_sc_gather():
    mesh = plsc.VectorSubcoreMesh(core_axis_name="c", subcore_axis_name="s")

    @functools.partial(
        pl.kernel,
        mesh=mesh,
        out_type=jax.ShapeDtypeStruct((_N, _D), jnp.float32),
        scratch_types=[
            pltpu.VMEM((_BPW,), jnp.int32),
            pltpu.VMEM((_BPW, _D), jnp.float32),
            pltpu.SemaphoreType.DMA,
            pltpu.SemaphoreType.DMA,
        ],
    )
    def _sc_gather(table_hbm, idx_hbm, out_hbm, idx_v, rows_v, sem, wsem):
        wid = lax.axis_index("s") * 2 + lax.axis_index("c")
        base = wid * _BPW
        pltpu.sync_copy(idx_hbm.at[pl.ds(base, _BPW)], idx_v)
        c0 = pltpu.async_copy(table_hbm.at[idx_v.at[pl.ds(0, _GCH)]],
                              rows_v.at[pl.ds(0, _GCH)], sem)
        c1 = pltpu.async_copy(table_hbm.at[idx_v.at[pl.ds(_GCH, _GCH)]],
                              rows_v.at[pl.ds(_GCH, _GCH)], sem)
        c0.wait()
        w0 = pltpu.async_copy(rows_v.at[pl.ds(0, _GCH)],
                              out_hbm.at[pl.ds(base, _GCH)], wsem)
        c1.wait()
        w1 = pltpu.async_copy(rows_v.at[pl.ds(_GCH, _GCH)],
                              out_hbm.at[pl.ds(base + _GCH, _GCH)], wsem)
        w0.wait()
        w1.wait()

    return _sc_gather


def kernel(inputs, embed_weight, training):
    idx3, loss, cmt, perp = _tc_call(inputs, embed_weight)
    idx_flat = idx3.reshape(_N)
    qflat = _get_sc_gather()(embed_weight, idx_flat)
    quantized_out = jnp.transpose(qflat.reshape(_B, _T, _D), (0, 2, 1))
    enc_idx = idx_flat.reshape(_B, _T, 1).astype(jnp.int64)
    return (quantized_out, loss[0, 0], cmt[0, 0], enc_idx, perp[0, 0])
